# Initial kernel scaffold; baseline (speedup 1.0000x reference)
import jax
import jax.numpy as jnp
from jax.experimental import pallas as pl


def kernel(inputs, codebook):
    raise NotImplementedError("write your pallas kernel here")


# speedup vs baseline: 2.7851x; 2.7851x over previous
import functools

import jax
import jax.numpy as jnp
from jax import lax
from jax.experimental import pallas as pl
from jax.experimental.pallas import tpu as pltpu
from jax.experimental.pallas import tpu_sc as plsc

_NUM_CODES = 512
_DIM = 64
_T = 1024


def _tc_body(x_ref, cbt_ref, idx_ref, loss_ref):
    i = pl.program_id(0)
    x = x_ref[...]
    cbt = cbt_ref[...]
    xb = x.astype(jnp.bfloat16)
    cb = cbt.astype(jnp.bfloat16)
    xc = lax.dot_general(xb, cb, (((1,), (0,)), ((), ())),
                         preferred_element_type=jnp.float32)
    xn = jnp.sum(x * x, axis=1, keepdims=True)
    cn = jnp.sum(cbt * cbt, axis=0, keepdims=True)
    dist = (xn + cn) - 2.0 * xc
    rowmin = jnp.min(dist, axis=1, keepdims=True)
    iot = lax.broadcasted_iota(jnp.int32, dist.shape, 1)
    masked = jnp.where(dist == rowmin, iot, _NUM_CODES)
    idx = jnp.min(masked, axis=1, keepdims=True)
    idx_ref[0] = idx
    s = jnp.sum(rowmin, keepdims=True).reshape(1, 1)

    @pl.when(i == 0)
    def _():
        loss_ref[...] = s

    @pl.when(i != 0)
    def _():
        loss_ref[...] += s


def _tc_argmin_call(flat, cbt, interpret=False):
    n = flat.shape[0]
    nb = n // _T
    return pl.pallas_call(
        _tc_body,
        grid=(nb,),
        in_specs=[
            pl.BlockSpec((_T, _DIM), lambda i: (i, 0)),
            pl.BlockSpec((_DIM, _NUM_CODES), lambda i: (0, 0)),
        ],
        out_specs=[
            pl.BlockSpec((1, _T, 1), lambda i: (i, 0, 0)),
            pl.BlockSpec((1, 1), lambda i: (0, 0)),
        ],
        out_shape=[
            jax.ShapeDtypeStruct((nb, _T, 1), jnp.int32),
            jax.ShapeDtypeStruct((1, 1), jnp.float32),
        ],
        compiler_params=pltpu.CompilerParams(
            dimension_semantics=("arbitrary",)),
        interpret=interpret,
    )(flat, cbt)


_SC_CHUNK = 128


def _sc_gather_call(table, idx):
    n = idx.shape[0]
    mesh = plsc.VectorSubcoreMesh(core_axis_name="c", subcore_axis_name="s")
    nw = 32
    bpw = n // nw
    nch = bpw // _SC_CHUNK

    @functools.partial(
        pl.kernel,
        mesh=mesh,
        out_type=jax.ShapeDtypeStruct((n, _DIM), jnp.float32),
        scratch_types=[
            pltpu.VMEM((_SC_CHUNK,), jnp.int32),
            pltpu.VMEM((_SC_CHUNK, _DIM), jnp.float32),
            pltpu.SemaphoreType.DMA,
        ],
        compiler_params=pltpu.CompilerParams(use_tc_tiling_on_sc=False),
    )
    def gather_k(table_hbm, idx_hbm, out_hbm, idx_v, rows_v, sem):
        wid = lax.axis_index("s") * 2 + lax.axis_index("c")
        for t in range(nch):
            base = wid * bpw + t * _SC_CHUNK
            pltpu.sync_copy(idx_hbm.at[pl.ds(base, _SC_CHUNK)], idx_v)
            pltpu.async_copy(table_hbm.at[idx_v], rows_v, sem).wait()
            pltpu.sync_copy(rows_v, out_hbm.at[pl.ds(base, _SC_CHUNK)])

    return gather_k(table, idx)


def kernel(inputs, codebook):
    flat = inputs.reshape(-1, _DIM)
    cbt = codebook.T
    idx3, loss_sum = _tc_argmin_call(flat, cbt)
    idx = idx3.reshape(-1)
    table = codebook.astype(jnp.bfloat16).astype(jnp.float32)
    q = _sc_gather_call(table, idx)
    n_elems = flat.shape[0] * _DIM
    loss = loss_sum[0, 0] * jnp.float32(1.25 / n_elems)
    return loss, q.reshape(inputs.shape)


# speedup vs baseline: 4.0676x; 1.4605x over previous
import functools

import jax
import jax.numpy as jnp
from jax import lax
from jax.experimental import pallas as pl
from jax.experimental.pallas import tpu as pltpu
from jax.experimental.pallas import tpu_sc as plsc

_NUM_CODES = 512
_DIM = 64
_T = 1024


_BPS = 8


def _tc_body(xt_ref, c_ref, idx_ref, loss_ref):
    i = pl.program_id(0)
    c = c_ref[...]
    cb = (-2.0 * c).astype(jnp.bfloat16)
    cn = jnp.sum(c * c, axis=1, keepdims=True)
    s = None
    for k in range(_BPS):
        xk = xt_ref[k * _DIM:(k + 1) * _DIM, :]
        xkb = xk.astype(jnp.bfloat16)
        xc2 = lax.dot_general(cb, xkb, (((1,), (0,)), ((), ())),
                              preferred_element_type=jnp.float32)
        xn = jnp.sum(xk * xk, axis=0, keepdims=True)
        dist = (xn + cn) + xc2
        m = jnp.min(dist, axis=0, keepdims=True)
        iot = lax.broadcasted_iota(jnp.int32, dist.shape, 0)
        masked = jnp.where(dist == m, iot, _NUM_CODES)
        idxr = jnp.min(masked, axis=0, keepdims=True)
        idx_ref[k:k + 1, :] = idxr
        sk = jnp.sum(m, keepdims=True).reshape(1, 1)
        s = sk if s is None else s + sk

    @pl.when(i == 0)
    def _():
        loss_ref[...] = s

    @pl.when(i != 0)
    def _():
        loss_ref[...] += s


def _tc_argmin_call(xt, codebook, interpret=False):
    nbatch = xt.shape[0] // _DIM
    nsteps = nbatch // _BPS
    return pl.pallas_call(
        _tc_body,
        grid=(nsteps,),
        in_specs=[
            pl.BlockSpec((_BPS * _DIM, 1024), lambda i: (i, 0)),
            pl.BlockSpec((_NUM_CODES, _DIM), lambda i: (0, 0)),
        ],
        out_specs=[
            pl.BlockSpec((_BPS, 1024), lambda i: (i, 0)),
            pl.BlockSpec((1, 1), lambda i: (0, 0)),
        ],
        out_shape=[
            jax.ShapeDtypeStruct((nbatch, 1024), jnp.int32),
            jax.ShapeDtypeStruct((1, 1), jnp.float32),
        ],
        compiler_params=pltpu.CompilerParams(
            dimension_semantics=("arbitrary",)),
        interpret=interpret,
    )(xt, codebook)


_SC_CHUNK = 128
_NBUF = 4


def _sc_gather_call(table128, idx2d):
    n = idx2d.shape[0] * idx2d.shape[1]
    mesh = plsc.VectorSubcoreMesh(core_axis_name="c", subcore_axis_name="s")
    nw = 32
    bpw = n // nw
    nch = bpw // _SC_CHUNK
    rpw = idx2d.shape[0] // nw

    @functools.partial(
        pl.kernel,
        mesh=mesh,
        out_type=jax.ShapeDtypeStruct((n, _DIM), jnp.float32),
        scratch_types=[
            pltpu.VMEM((rpw, _SC_CHUNK), jnp.int32),
            [pltpu.VMEM((_SC_CHUNK, _DIM), jnp.float32)] * _NBUF,
            pltpu.SemaphoreType.DMA,
            [pltpu.SemaphoreType.DMA] * _NBUF,
            [pltpu.SemaphoreType.DMA] * _NBUF,
        ],
        compiler_params=pltpu.CompilerParams(use_tc_tiling_on_sc=False),
    )
    def gather_k(table_hbm, idx_hbm, out_hbm, idx_v, rows_v, isem, gsems, ssems):
        wid = lax.axis_index("s") * 2 + lax.axis_index("c")
        pltpu.async_copy(idx_hbm.at[pl.ds(wid * rpw, rpw)], idx_v, isem).wait()
        gathers = [None] * nch
        scatters = [None] * nch
        for t in range(nch):
            b = t % _NBUF
            if t >= _NBUF:
                scatters[t - _NBUF].wait()
            gathers[t] = pltpu.async_copy(
                table_hbm.at[idx_v.at[t]], rows_v[b], gsems[b])
            if t >= 1:
                gathers[t - 1].wait()
                tp = t - 1
                bp = tp % _NBUF
                base = wid * bpw + tp * _SC_CHUNK
                scatters[tp] = pltpu.async_copy(
                    rows_v[bp], out_hbm.at[pl.ds(base, _SC_CHUNK)], ssems[bp])
        t = nch - 1
        gathers[t].wait()
        base = wid * bpw + t * _SC_CHUNK
        scatters[t] = pltpu.async_copy(
            rows_v[t % _NBUF], out_hbm.at[pl.ds(base, _SC_CHUNK)],
            ssems[t % _NBUF])
        for t in range(nch - _NBUF, nch):
            scatters[t].wait()

    return gather_k(table128, idx2d)


def kernel(inputs, codebook):
    xt = jnp.swapaxes(inputs, 1, 2).reshape(-1, inputs.shape[1])
    idx64, loss_sum = _tc_argmin_call(xt, codebook)
    idx2d = idx64.reshape(-1, 128)
    table = codebook.astype(jnp.bfloat16).astype(jnp.float32)
    q = _sc_gather_call(table, idx2d)
    n_elems = idx64.size * _DIM
    loss = loss_sum[0, 0] * jnp.float32(1.25 / n_elems)
    return loss, q.reshape(inputs.shape)


# speedup vs baseline: 5.3066x; 1.3046x over previous
import functools

import jax
import jax.numpy as jnp
from jax import lax
from jax.experimental import pallas as pl
from jax.experimental.pallas import tpu as pltpu
from jax.experimental.pallas import tpu_sc as plsc

_NUM_CODES = 512
_DIM = 64
_T = 1024


_BPS = 8


def _tc_body(xt_ref, c_ref, idx_ref, loss_ref, tbl_ref):
    i = pl.program_id(0)
    c = c_ref[...]
    cb = (-2.0 * c).astype(jnp.bfloat16)
    cn = jnp.sum(c * c, axis=1, keepdims=True)

    @pl.when(i == 0)
    def _():
        rounded = cb.astype(jnp.float32) * -0.5
        tbl_ref[...] = lax.transpose(rounded, (1, 0))

    s = None
    for k in range(_BPS):
        xk = xt_ref[k * _DIM:(k + 1) * _DIM, :]
        xkb = xk.astype(jnp.bfloat16)
        xc2 = lax.dot_general(cb, xkb, (((1,), (0,)), ((), ())),
                              preferred_element_type=jnp.float32)
        xn = jnp.sum(xk * xk, axis=0, keepdims=True)
        dist = (xn + cn) + xc2
        m = jnp.min(dist, axis=0, keepdims=True)
        iot = lax.broadcasted_iota(jnp.int32, dist.shape, 0)
        masked = jnp.where(dist == m, iot, _NUM_CODES)
        idxr = jnp.min(masked, axis=0, keepdims=True)
        idx_ref[k:k + 1, :] = idxr
        sk = jnp.sum(m, keepdims=True).reshape(1, 1)
        s = sk if s is None else s + sk

    @pl.when(i == 0)
    def _():
        loss_ref[...] = s

    @pl.when(i != 0)
    def _():
        loss_ref[...] += s


def _tc_argmin_call(xt, codebook, interpret=False):
    nbatch = xt.shape[0] // _DIM
    nsteps = nbatch // _BPS
    return pl.pallas_call(
        _tc_body,
        grid=(nsteps,),
        in_specs=[
            pl.BlockSpec((_BPS * _DIM, 1024), lambda i: (i, 0)),
            pl.BlockSpec((_NUM_CODES, _DIM), lambda i: (0, 0)),
        ],
        out_specs=[
            pl.BlockSpec((_BPS, 1024), lambda i: (i, 0)),
            pl.BlockSpec((1, 1), lambda i: (0, 0)),
            pl.BlockSpec((_DIM, _NUM_CODES), lambda i: (0, 0)),
        ],
        out_shape=[
            jax.ShapeDtypeStruct((nbatch, 1024), jnp.int32),
            jax.ShapeDtypeStruct((1, 1), jnp.float32),
            jax.ShapeDtypeStruct((_DIM, _NUM_CODES), jnp.float32),
        ],
        compiler_params=pltpu.CompilerParams(
            dimension_semantics=("arbitrary",)),
        interpret=interpret,
    )(xt, codebook)


_TPW = 256
_GRP = 16


def _sc_gather_call(tableT, idx64):
    nbatch, ntok = idx64.shape
    mesh = plsc.VectorSubcoreMesh(core_axis_name="c", subcore_axis_name="s")
    ncg = ntok // _TPW
    nband = 32 // ncg
    bpb = nbatch // nband

    @functools.partial(
        pl.kernel,
        mesh=mesh,
        out_type=jax.ShapeDtypeStruct((nbatch * _DIM, ntok), jnp.float32),
        scratch_types=[
            pltpu.VMEM((_DIM, _NUM_CODES), jnp.float32),
            pltpu.VMEM((bpb, _TPW), jnp.int32),
            [pltpu.VMEM((_DIM, _TPW), jnp.float32)] * 2,
            pltpu.SemaphoreType.DMA,
            pltpu.SemaphoreType.DMA,
            [pltpu.SemaphoreType.DMA] * 2,
        ],
        compiler_params=pltpu.CompilerParams(needs_layout_passes=False),
    )
    def gather_k(table_hbm, idx_hbm, out_hbm, table_v, idx_v, q_v, tsem, isem,
                 ssems):
        wid = lax.axis_index("s") * 2 + lax.axis_index("c")
        band = wid // ncg
        cg = lax.rem(wid, ncg)
        tcopy = pltpu.async_copy(table_hbm, table_v, tsem)
        pltpu.async_copy(
            idx_hbm.at[pl.ds(band * bpb, bpb), pl.ds(cg * _TPW, _TPW)],
            idx_v, isem).wait()
        tcopy.wait()
        scatters = [None, None]
        for bb in range(bpb):
            buf = bb % 2
            if scatters[buf] is not None:
                scatters[buf].wait()

            def body(g, _, bb=bb, buf=buf):
                idx16 = idx_v[bb, pl.ds(g * _GRP, _GRP)]
                for d in range(_DIM):
                    q_v[buf][d, pl.ds(g * _GRP, _GRP)] = plsc.load_gather(
                        table_v, [jnp.full((_GRP,), d, jnp.int32), idx16])
                return _

            lax.fori_loop(0, _TPW // _GRP, body, 0)
            scatters[buf] = pltpu.async_copy(
                q_v[buf],
                out_hbm.at[pl.ds((band * bpb + bb) * _DIM, _DIM),
                           pl.ds(cg * _TPW, _TPW)],
                ssems[buf])
        scatters[0].wait()
        scatters[1].wait()

    return gather_k(tableT, idx64)


def kernel(inputs, codebook):
    xt = jnp.swapaxes(inputs, 1, 2).reshape(-1, inputs.shape[1])
    idx64, loss_sum, tableT = _tc_argmin_call(xt, codebook)
    qt = _sc_gather_call(tableT, idx64)
    n_elems = idx64.size * _DIM
    loss = loss_sum[0, 0] * jnp.float32(1.25 / n_elems)
    nb = inputs.shape[0]
    q = jnp.swapaxes(qt.reshape(nb, _DIM, inputs.shape[1]), 1, 2)
    return loss, q


# speedup vs baseline: 6.3664x; 1.1997x over previous
import functools

import jax
import jax.numpy as jnp
from jax import lax
from jax.experimental import pallas as pl
from jax.experimental.pallas import tpu as pltpu
from jax.experimental.pallas import tpu_sc as plsc

_NUM_CODES = 512
_DIM = 64
_T = 1024


_BPS = 8


def _tc_body(xt_ref, c_ref, idx_ref, loss_ref, tbl_ref):
    i = pl.program_id(0)
    c = c_ref[...]
    cb = (-2.0 * c).astype(jnp.bfloat16)
    cn = jnp.sum(c * c, axis=1, keepdims=True)

    @pl.when(i == 0)
    def _():
        rounded = cb.astype(jnp.float32) * -0.5
        tbl_ref[...] = lax.transpose(rounded, (1, 0))

    s = None
    for k in range(_BPS):
        xk = xt_ref[k * _DIM:(k + 1) * _DIM, :]
        xkb = xk.astype(jnp.bfloat16)
        xc2 = lax.dot_general(cb, xkb, (((1,), (0,)), ((), ())),
                              preferred_element_type=jnp.float32)
        xn = jnp.sum(xk * xk, axis=0, keepdims=True)
        dist = (xn + cn) + xc2
        m = jnp.min(dist, axis=0, keepdims=True)
        iot = lax.broadcasted_iota(jnp.int32, dist.shape, 0)
        masked = jnp.where(dist == m, iot, _NUM_CODES)
        idxr = jnp.min(masked, axis=0, keepdims=True)
        idx_ref[k:k + 1, :] = idxr
        sk = jnp.sum(m, keepdims=True).reshape(1, 1)
        s = sk if s is None else s + sk

    @pl.when(i == 0)
    def _():
        loss_ref[...] = s

    @pl.when(i != 0)
    def _():
        loss_ref[...] += s


def _tc_argmin_call(xt, codebook, interpret=False):
    nbatch = xt.shape[0] // _DIM
    nsteps = nbatch // _BPS
    return pl.pallas_call(
        _tc_body,
        grid=(nsteps,),
        in_specs=[
            pl.BlockSpec((_BPS * _DIM, 1024), lambda i: (i, 0)),
            pl.BlockSpec((_NUM_CODES, _DIM), lambda i: (0, 0)),
        ],
        out_specs=[
            pl.BlockSpec((_BPS, 1024), lambda i: (i, 0)),
            pl.BlockSpec((1, 1), lambda i: (0, 0)),
            pl.BlockSpec((_DIM, _NUM_CODES), lambda i: (0, 0)),
        ],
        out_shape=[
            jax.ShapeDtypeStruct((nbatch, 1024), jnp.int32),
            jax.ShapeDtypeStruct((1, 1), jnp.float32),
            jax.ShapeDtypeStruct((_DIM, _NUM_CODES), jnp.float32),
        ],
        compiler_params=pltpu.CompilerParams(
            dimension_semantics=("arbitrary",)),
        interpret=interpret,
    )(xt, codebook)


_TPW = 256
_GRP = 16


def _sc_gather_call(tableT, idx64):
    nbatch, ntok = idx64.shape
    mesh = plsc.VectorSubcoreMesh(core_axis_name="c", subcore_axis_name="s")
    ncg = ntok // _TPW
    nband = 32 // ncg
    bpb = nbatch // nband

    @functools.partial(
        pl.kernel,
        mesh=mesh,
        out_type=jax.ShapeDtypeStruct((nbatch * _DIM, ntok), jnp.float32),
        scratch_types=[
            pltpu.VMEM((_DIM, _NUM_CODES), jnp.float32),
            pltpu.VMEM((bpb, _TPW), jnp.int32),
            [pltpu.VMEM((_DIM, _TPW), jnp.float32)] * 2,
            pltpu.SemaphoreType.DMA,
            pltpu.SemaphoreType.DMA,
            [pltpu.SemaphoreType.DMA] * 2,
        ],
        compiler_params=pltpu.CompilerParams(needs_layout_passes=False),
    )
    def gather_k(table_hbm, idx_hbm, out_hbm, table_v, idx_v, q_v, tsem, isem,
                 ssems):
        wid = lax.axis_index("s") * 2 + lax.axis_index("c")
        band = wid // ncg
        cg = lax.rem(wid, ncg)
        tcopy = pltpu.async_copy(table_hbm, table_v, tsem)
        pltpu.async_copy(
            idx_hbm.at[pl.ds(band * bpb, bpb), pl.ds(cg * _TPW, _TPW)],
            idx_v, isem).wait()
        tcopy.wait()
        scatters = [None, None]
        for bb in range(bpb):
            buf = bb % 2
            if scatters[buf] is not None:
                scatters[buf].wait()

            @plsc.parallel_loop(0, _TPW // _GRP)
            def body(g, bb=bb, buf=buf):
                idx16 = idx_v[bb, pl.ds(g * _GRP, _GRP)]
                for d in range(_DIM):
                    q_v[buf][d, pl.ds(g * _GRP, _GRP)] = plsc.load_gather(
                        table_v, [jnp.full((_GRP,), d, jnp.int32), idx16])
            scatters[buf] = pltpu.async_copy(
                q_v[buf],
                out_hbm.at[pl.ds((band * bpb + bb) * _DIM, _DIM),
                           pl.ds(cg * _TPW, _TPW)],
                ssems[buf])
        scatters[0].wait()
        scatters[1].wait()

    return gather_k(tableT, idx64)


def kernel(inputs, codebook):
    xt = jnp.swapaxes(inputs, 1, 2).reshape(-1, inputs.shape[1])
    idx64, loss_sum, tableT = _tc_argmin_call(xt, codebook)
    qt = _sc_gather_call(tableT, idx64)
    n_elems = idx64.size * _DIM
    loss = loss_sum[0, 0] * jnp.float32(1.25 / n_elems)
    nb = inputs.shape[0]
    q = jnp.swapaxes(qt.reshape(nb, _DIM, inputs.shape[1]), 1, 2)
    return loss, q


# speedup vs baseline: 7.1238x; 1.1190x over previous
import functools

import jax
import jax.numpy as jnp
from jax import lax
from jax.experimental import pallas as pl
from jax.experimental.pallas import tpu as pltpu
from jax.experimental.pallas import tpu_sc as plsc

_NUM_CODES = 512
_DIM = 64
_T = 1024


_BPS = 8


def _tc_body(xt_ref, c_ref, idx_ref, loss_ref, tbl_ref):
    i = pl.program_id(0)
    c = c_ref[...]
    cb = (-2.0 * c).astype(jnp.bfloat16)
    cn = jnp.sum(c * c, axis=1, keepdims=True)

    @pl.when(i == 0)
    def _():
        rounded = cb.astype(jnp.float32) * -0.5
        tbl_ref[...] = lax.transpose(rounded, (1, 0))

    iota8 = lax.broadcasted_iota(jnp.int32, (8, 1024), 0)
    s = None
    for k in range(_BPS):
        xk = xt_ref[k * _DIM:(k + 1) * _DIM, :]
        xkb = xk.astype(jnp.bfloat16)
        xc2 = lax.dot_general(cb, xkb, (((1,), (0,)), ((), ())),
                              preferred_element_type=jnp.float32)
        xn = jnp.sum(xk * xk, axis=0, keepdims=True)
        best = bidx = None
        for r in range(_NUM_CODES // 8):
            d_r = (xn + cn[r * 8:(r + 1) * 8]) + xc2[r * 8:(r + 1) * 8]
            i_r = iota8 + (r * 8)
            if best is None:
                best, bidx = d_r, i_r
            else:
                pred = d_r < best
                best = jnp.where(pred, d_r, best)
                bidx = jnp.where(pred, i_r, bidx)
        m = jnp.min(best, axis=0, keepdims=True)
        cand = jnp.where(best == m, bidx, _NUM_CODES)
        idxr = jnp.min(cand, axis=0, keepdims=True)
        idx_ref[k:k + 1, :] = idxr
        sk = jnp.sum(m, keepdims=True).reshape(1, 1)
        s = sk if s is None else s + sk

    @pl.when(i == 0)
    def _():
        loss_ref[...] = s

    @pl.when(i != 0)
    def _():
        loss_ref[...] += s


def _tc_argmin_call(xt, codebook, interpret=False):
    nbatch = xt.shape[0] // _DIM
    nsteps = nbatch // _BPS
    return pl.pallas_call(
        _tc_body,
        grid=(nsteps,),
        in_specs=[
            pl.BlockSpec((_BPS * _DIM, 1024), lambda i: (i, 0)),
            pl.BlockSpec((_NUM_CODES, _DIM), lambda i: (0, 0)),
        ],
        out_specs=[
            pl.BlockSpec((_BPS, 1024), lambda i: (i, 0)),
            pl.BlockSpec((1, 1), lambda i: (0, 0)),
            pl.BlockSpec((_DIM, _NUM_CODES), lambda i: (0, 0)),
        ],
        out_shape=[
            jax.ShapeDtypeStruct((nbatch, 1024), jnp.int32),
            jax.ShapeDtypeStruct((1, 1), jnp.float32),
            jax.ShapeDtypeStruct((_DIM, _NUM_CODES), jnp.float32),
        ],
        compiler_params=pltpu.CompilerParams(
            dimension_semantics=("arbitrary",)),
        interpret=interpret,
    )(xt, codebook)


_TPW = 256
_GRP = 16


def _sc_gather_call(tableT, idx64):
    nbatch, ntok = idx64.shape
    mesh = plsc.VectorSubcoreMesh(core_axis_name="c", subcore_axis_name="s")
    ncg = ntok // _TPW
    nband = 32 // ncg
    bpb = nbatch // nband

    @functools.partial(
        pl.kernel,
        mesh=mesh,
        out_type=jax.ShapeDtypeStruct((nbatch * _DIM, ntok), jnp.float32),
        scratch_types=[
            pltpu.VMEM((_DIM, _NUM_CODES), jnp.float32),
            pltpu.VMEM((bpb, _TPW), jnp.int32),
            [pltpu.VMEM((_DIM, _TPW), jnp.float32)] * 2,
            pltpu.SemaphoreType.DMA,
            pltpu.SemaphoreType.DMA,
            [pltpu.SemaphoreType.DMA] * 2,
        ],
        compiler_params=pltpu.CompilerParams(needs_layout_passes=False),
    )
    def gather_k(table_hbm, idx_hbm, out_hbm, table_v, idx_v, q_v, tsem, isem,
                 ssems):
        wid = lax.axis_index("s") * 2 + lax.axis_index("c")
        band = wid // ncg
        cg = lax.rem(wid, ncg)
        tcopy = pltpu.async_copy(table_hbm, table_v, tsem)
        pltpu.async_copy(
            idx_hbm.at[pl.ds(band * bpb, bpb), pl.ds(cg * _TPW, _TPW)],
            idx_v, isem).wait()
        tcopy.wait()
        scatters = [None, None]
        for bb in range(bpb):
            buf = bb % 2
            if scatters[buf] is not None:
                scatters[buf].wait()

            @plsc.parallel_loop(0, _TPW // _GRP)
            def body(g, bb=bb, buf=buf):
                idx16 = idx_v[bb, pl.ds(g * _GRP, _GRP)]
                for d in range(_DIM):
                    q_v[buf][d, pl.ds(g * _GRP, _GRP)] = plsc.load_gather(
                        table_v, [jnp.full((_GRP,), d, jnp.int32), idx16])
            scatters[buf] = pltpu.async_copy(
                q_v[buf],
                out_hbm.at[pl.ds((band * bpb + bb) * _DIM, _DIM),
                           pl.ds(cg * _TPW, _TPW)],
                ssems[buf])
        scatters[0].wait()
        scatters[1].wait()

    return gather_k(tableT, idx64)


def kernel(inputs, codebook):
    xt = jnp.swapaxes(inputs, 1, 2).reshape(-1, inputs.shape[1])
    idx64, loss_sum, tableT = _tc_argmin_call(xt, codebook)
    qt = _sc_gather_call(tableT, idx64)
    n_elems = idx64.size * _DIM
    loss = loss_sum[0, 0] * jnp.float32(1.25 / n_elems)
    nb = inputs.shape[0]
    q = jnp.swapaxes(qt.reshape(nb, _DIM, inputs.shape[1]), 1, 2)
    return loss, q
